# baseline (speedup 1.0000x reference)
import functools

import jax
import jax.numpy as jnp
from jax import lax
from jax.experimental import pallas as pl
from jax.experimental.pallas import tpu as pltpu
from jax.experimental.pallas import tpu_sc as plsc

_TC_INTERPRET = False


def _tc_inproj(x, W_in, b_in):
    N, D = x.shape
    H = W_in.shape[1]
    NB = 2000
    Hh = H // 2

    def kfn(x_ref, w_ref, b_ref, h_ref):
        acc = jnp.dot(x_ref[...], w_ref[...],
                      preferred_element_type=jnp.float32) + b_ref[...]
        h_ref[0] = acc[:, :Hh]
        h_ref[1] = acc[:, Hh:]

    return pl.pallas_call(
        kfn,
        grid=(N // NB,),
        in_specs=[pl.BlockSpec((NB, D), lambda i: (i, 0)),
                  pl.BlockSpec((D, H), lambda i: (0, 0)),
                  pl.BlockSpec((1, H), lambda i: (0, 0))],
        out_specs=pl.BlockSpec((2, NB, Hh), lambda i: (0, i, 0)),
        out_shape=jax.ShapeDtypeStruct((2, N, Hh), jnp.float32),
        interpret=_TC_INTERPRET,
    )(x, W_in, b_in)


def _tc_edgeproj(eat, W_e, b_e):
    De, Ep = eat.shape
    H = W_e.shape[1]
    Hh = H // 2
    Q = Ep // 4
    EB = 3200
    nblk = Q // EB

    def kfn(a0, a1, a2, a3, wa_ref, wb_ref, ba_ref, bb_ref, o_ref):
        wa = wa_ref[...]
        wb = wb_ref[...]
        dn = (((0,), (0,)), ((), ()))
        for k, a_ref in enumerate((a0, a1, a2, a3)):
            a = a_ref[...]
            o_ref[0, :, k * Hh:(k + 1) * Hh] = lax.dot_general(
                a, wa, dn, preferred_element_type=jnp.float32) + ba_ref[...]
            o_ref[1, :, k * Hh:(k + 1) * Hh] = lax.dot_general(
                a, wb, dn, preferred_element_type=jnp.float32) + bb_ref[...]

    def mk_spec(k):
        return pl.BlockSpec((De, EB), lambda i, k=k: (0, k * nblk + i))

    return pl.pallas_call(
        kfn,
        grid=(nblk,),
        in_specs=[mk_spec(0), mk_spec(1), mk_spec(2), mk_spec(3),
                  pl.BlockSpec((De, Hh), lambda i: (0, 0)),
                  pl.BlockSpec((De, Hh), lambda i: (0, 0)),
                  pl.BlockSpec((1, Hh), lambda i: (0, 0)),
                  pl.BlockSpec((1, Hh), lambda i: (0, 0))],
        out_specs=pl.BlockSpec((2, EB, 4 * Hh), lambda i: (0, i, 0)),
        out_shape=jax.ShapeDtypeStruct((2, Q, 4 * Hh), jnp.float32),
        interpret=_TC_INTERPRET,
    )(eat, eat, eat, eat, W_e[:, :Hh], W_e[:, Hh:],
      b_e[:Hh].reshape(1, Hh), b_e[Hh:].reshape(1, Hh))


def _tc_mlp(h2, g2, W1, b1, W2, b2, mid):
    _, N, Hh = h2.shape
    H = 2 * Hh
    NB = 2000

    def kfn(h_ref, g_ref, w1_ref, b1_ref, w2_ref, b2_ref, o_ref):
        hi0 = h_ref[0] + g_ref[0]
        hi1 = h_ref[1] + g_ref[1]
        w1 = w1_ref[...]
        t = jnp.dot(hi0, w1[:Hh, :], preferred_element_type=jnp.float32)
        t = t + jnp.dot(hi1, w1[Hh:, :], preferred_element_type=jnp.float32)
        t = jnp.maximum(t + b1_ref[...], 0.0)
        o = jnp.dot(t, w2_ref[...], preferred_element_type=jnp.float32)
        o = o + b2_ref[...]
        if mid:
            o = jnp.maximum(o, 0.0)
            o_ref[0] = o[:, :Hh]
            o_ref[1] = o[:, Hh:]
        else:
            o_ref[...] = o

    if mid:
        out_specs = pl.BlockSpec((2, NB, Hh), lambda i: (0, i, 0))
        out_shape = jax.ShapeDtypeStruct((2, N, Hh), jnp.float32)
    else:
        out_specs = pl.BlockSpec((NB, H), lambda i: (i, 0))
        out_shape = jax.ShapeDtypeStruct((N, H), jnp.float32)

    return pl.pallas_call(
        kfn,
        grid=(N // NB,),
        in_specs=[pl.BlockSpec((2, NB, Hh), lambda i: (0, i, 0)),
                  pl.BlockSpec((2, NB, Hh), lambda i: (0, i, 0)),
                  pl.BlockSpec((H, H), lambda i: (0, 0)),
                  pl.BlockSpec((1, H), lambda i: (0, 0)),
                  pl.BlockSpec((H, H), lambda i: (0, 0)),
                  pl.BlockSpec((1, H), lambda i: (0, 0))],
        out_specs=out_specs,
        out_shape=out_shape,
        interpret=_TC_INTERPRET,
    )(h2, g2, W1, b1, W2, b2)


def _sc_aggr(h2, ea2, src2, dst2, E_real):
    _, N, Hh = h2.shape
    nchunks, K = src2.shape
    Q = ea2.shape[1]
    T = 16
    base_chunks = nchunks // T
    G = 32
    NGRP = -(-base_chunks // G)
    RPT = 3200
    ZR = 80
    nz_last = (N - 15 * RPT) // ZR
    nz_main = RPT // ZR
    KP = K // 4
    FAKE_C0 = (E_real - 3 * (nchunks * 32)) // 32

    mesh = plsc.VectorSubcoreMesh(core_axis_name="c", subcore_axis_name="s",
                                  num_cores=2, num_subcores=T)

    @functools.partial(
        pl.kernel,
        out_type=jax.ShapeDtypeStruct((2, N, Hh), jnp.float32),
        mesh=mesh,
        scratch_types=[
            pltpu.VMEM((G, K), jnp.int32),
            pltpu.VMEM((G, K), jnp.int32),
            pltpu.VMEM((2, K, Hh), jnp.float32),
            pltpu.VMEM((2, KP, 128), jnp.float32),
            pltpu.VMEM((ZR, Hh), jnp.float32),
            pltpu.VMEM_SHARED((N, Hh), jnp.float32),
            pltpu.SemaphoreType.DMA,
            pltpu.SemaphoreType.DMA,
            pltpu.SemaphoreType.DMA,
            pltpu.SemaphoreType.DMA,
            pltpu.SemaphoreType.DMA,
            pltpu.SemaphoreType.DMA,
        ],
        compiler_params=pltpu.CompilerParams(use_tc_tiling_on_sc=False),
    )
    def k(h_ref, ea_ref, src_ref, dst_ref, out_ref,
          sgrp, dgrp, gath, eabuf, stage, aggr,
          sg0, sg1, se0, se1, ss0, ss1):
        c = lax.axis_index("c")
        s = lax.axis_index("s")
        sem_g = (sg0, sg1)
        sem_e = (se0, se1)
        sem_s = (ss0, ss1)
        ht = h_ref.at[c]
        eac = ea_ref.at[c]
        outc = out_ref.at[c]

        def zrow(r, carry):
            z = jnp.zeros((16,), jnp.float32)
            for q in range(Hh // 16):
                stage[r, pl.ds(q * 16, 16)] = z
            return carry
        lax.fori_loop(0, ZR, zrow, 0)
        row0 = s * RPT
        nzs = jnp.where(s < 15, nz_main, nz_last)

        def zchunk(t, carry):
            r0 = pl.multiple_of(row0 + t * ZR, 8)
            pltpu.sync_copy(stage, aggr.at[pl.ds(r0, ZR), :])
            return carry
        lax.fori_loop(0, nzs, zchunk, 0)
        plsc.subcore_barrier()

        start = s * base_chunks
        def gather_desc(slot, row):
            return pltpu.make_async_copy(
                ht.at[sgrp.at[row]], gath.at[slot], sem_g[slot])

        def ea_desc(slot, j):
            p0 = pl.multiple_of(j * KP, KP)
            return pltpu.make_async_copy(
                eac.at[pl.ds(p0, KP), :], eabuf.at[slot], sem_e[slot])

        def scat_desc(slot, row):
            return pltpu.make_async_copy(
                gath.at[slot], aggr.at[dgrp.at[row]], sem_s[slot])

        def issue(slot, row, j, pend):
            @pl.when(pend == 1)
            def _():
                scat_desc(slot, row).wait()
            gather_desc(slot, row).start()
            ea_desc(slot, j).start()

        def process(slot, row, j):
            gather_desc(slot, row).wait()
            ea_desc(slot, 0).wait()

            def prow(pr, carry):
                for sub in range(8):
                    e = 32 * (sub // 2) + pr
                    f0 = (sub % 2) * 16
                    a = gath[slot, e, pl.ds(f0, 16)]
                    b = eabuf[slot, pr, pl.ds(sub * 16, 16)]
                    gath[slot, e, pl.ds(f0, 16)] = jnp.maximum(a + b, 0.0)
                return carry
            lax.fori_loop(0, KP, prow, 0)

            @pl.when(j >= FAKE_C0)
            def _():
                z = jnp.zeros((16,), jnp.float32)

                def zfake(r, carry):
                    gath[slot, r, pl.ds(0, 16)] = z
                    gath[slot, r, pl.ds(16, 16)] = z
                    return carry
                lax.fori_loop(96, 128, zfake, 0)
            scat_desc(slot, row).start(add=True)

        pend0 = jnp.int32(0)
        pend1 = jnp.int32(0)
        for g in range(NGRP):
            gbase = start + g * G
            gcnt = min(G, base_chunks - g * G)
            pltpu.sync_copy(src_ref.at[pl.ds(gbase, G), :], sgrp)
            pltpu.sync_copy(dst_ref.at[pl.ds(gbase, G), :], dgrp)

            issue(0, 0, gbase, pend0)
            npairs = (gcnt + 1) // 2

            def pair(p, carry):
                p0, p1 = carry
                jj0 = 2 * p
                jj1 = jj0 + 1

                @pl.when(jj1 < gcnt)
                def _():
                    issue(1, jj1, gbase + jj1, p1)
                process(0, jj0, gbase + jj0)
                new_p0 = jnp.int32(1)

                @pl.when(jj1 + 1 < gcnt)
                def _():
                    issue(0, jj1 + 1, gbase + jj1 + 1, new_p0)

                @pl.when(jj1 < gcnt)
                def _():
                    process(1, jj1, gbase + jj1)
                new_p1 = jnp.where(jj1 < gcnt, jnp.int32(1), p1)
                return (new_p0, new_p1)

            pend0, pend1 = lax.fori_loop(0, npairs, pair, (pend0, pend1))

        @pl.when(pend0 == 1)
        def _():
            scat_desc(0, 0).wait()

        @pl.when(pend1 == 1)
        def _():
            scat_desc(1, 0).wait()

        plsc.subcore_barrier()

        def wchunk(t, carry):
            r0 = pl.multiple_of(row0 + t * ZR, 8)
            pltpu.sync_copy(aggr.at[pl.ds(r0, ZR), :], stage)
            pltpu.sync_copy(stage, outc.at[pl.ds(r0, ZR), :])
            return carry
        lax.fori_loop(0, nzs, wchunk, 0)

    return k(h2, ea2, src2, dst2)


def kernel(x, edge_index, edge_attr, W_in, b_in, W_e, b_e, W1, b1, W2, b2):
    E = edge_attr.shape[0]
    De = edge_attr.shape[1]
    N = x.shape[0]
    Ep = 819200
    eat = jnp.concatenate(
        [edge_attr.T, jnp.zeros((De, Ep - E), jnp.float32)], axis=1)
    srcp = jnp.concatenate(
        [edge_index[0], jnp.zeros((Ep - E,), jnp.int32)])
    dstp = jnp.concatenate(
        [edge_index[1], jnp.arange(Ep - E, dtype=jnp.int32) % N])
    src2 = srcp.reshape(4, Ep // 128, 32).transpose(1, 0, 2)
    src2 = src2.reshape(Ep // 128, 128)
    dst2 = dstp.reshape(4, Ep // 128, 32).transpose(1, 0, 2)
    dst2 = dst2.reshape(Ep // 128, 128)
    b_in2 = b_in.reshape(1, -1)
    b12 = b1.reshape(1, -1)
    b22 = b2.reshape(1, -1)

    h0 = _tc_inproj(x, W_in, b_in2)
    ea2 = _tc_edgeproj(eat, W_e, b_e)

    g2 = _sc_aggr(h0, ea2, src2, dst2, E)
    h1 = _tc_mlp(h0, g2, W1, b12, W2, b22, mid=True)

    g2 = _sc_aggr(h1, ea2, src2, dst2, E)
    out = _tc_mlp(h1, g2, W1, b12, W2, b22, mid=False)
    return out


# speedup vs baseline: 1.4027x; 1.4027x over previous
import functools

import jax
import jax.numpy as jnp
from jax import lax
from jax.experimental import pallas as pl
from jax.experimental.pallas import tpu as pltpu
from jax.experimental.pallas import tpu_sc as plsc

_TC_INTERPRET = False


def _tc_inproj(x, W_in, b_in):
    N, D = x.shape
    H = W_in.shape[1]
    NB = 2000
    Hh = H // 2

    def kfn(x_ref, w_ref, b_ref, h0_ref, h1_ref):
        acc = jnp.dot(x_ref[...], w_ref[...],
                      preferred_element_type=jnp.float32) + b_ref[...]
        h0_ref[...] = acc[:, :Hh]
        h1_ref[...] = acc[:, Hh:]

    return pl.pallas_call(
        kfn,
        grid=(N // NB,),
        in_specs=[pl.BlockSpec((NB, D), lambda i: (i, 0)),
                  pl.BlockSpec((D, H), lambda i: (0, 0)),
                  pl.BlockSpec((1, H), lambda i: (0, 0))],
        out_specs=[pl.BlockSpec((NB, Hh), lambda i: (i, 0)),
                   pl.BlockSpec((NB, Hh), lambda i: (i, 0))],
        out_shape=[jax.ShapeDtypeStruct((N, Hh), jnp.float32),
                   jax.ShapeDtypeStruct((N, Hh), jnp.float32)],
        interpret=_TC_INTERPRET,
    )(x, W_in, b_in)


def _tc_edgeproj(ea4, W4a, W4b, b4a, b4b):
    G4, _ = ea4.shape
    EB = 1000

    def kfn(a_ref, wa_ref, wb_ref, ba_ref, bb_ref, o0_ref, o1_ref):
        a = a_ref[...]
        o0_ref[...] = jnp.dot(a, wa_ref[...],
                              preferred_element_type=jnp.float32) + ba_ref[...]
        o1_ref[...] = jnp.dot(a, wb_ref[...],
                              preferred_element_type=jnp.float32) + bb_ref[...]

    return pl.pallas_call(
        kfn,
        grid=(G4 // EB,),
        in_specs=[pl.BlockSpec((EB, 64), lambda i: (i, 0)),
                  pl.BlockSpec((64, 128), lambda i: (0, 0)),
                  pl.BlockSpec((64, 128), lambda i: (0, 0)),
                  pl.BlockSpec((1, 128), lambda i: (0, 0)),
                  pl.BlockSpec((1, 128), lambda i: (0, 0))],
        out_specs=[pl.BlockSpec((EB, 128), lambda i: (i, 0)),
                   pl.BlockSpec((EB, 128), lambda i: (i, 0))],
        out_shape=[jax.ShapeDtypeStruct((G4, 128), jnp.float32),
                   jax.ShapeDtypeStruct((G4, 128), jnp.float32)],
        interpret=_TC_INTERPRET,
    )(ea4, W4a, W4b, b4a, b4b)


def _tc_mlp(ha, hb, ga, gb, W1, b1, W2, b2, mid):
    N, Hh = ha.shape
    H = 2 * Hh
    NB = 2000

    def kfn(ha_ref, hb_ref, ga_ref, gb_ref, w1_ref, b1_ref, w2_ref, b2_ref,
            *outs):
        hi0 = ha_ref[...] + ga_ref[...]
        hi1 = hb_ref[...] + gb_ref[...]
        w1 = w1_ref[...]
        t = jnp.dot(hi0, w1[:Hh, :], preferred_element_type=jnp.float32)
        t = t + jnp.dot(hi1, w1[Hh:, :], preferred_element_type=jnp.float32)
        t = jnp.maximum(t + b1_ref[...], 0.0)
        o = jnp.dot(t, w2_ref[...], preferred_element_type=jnp.float32)
        o = o + b2_ref[...]
        if mid:
            o = jnp.maximum(o, 0.0)
            outs[0][...] = o[:, :Hh]
            outs[1][...] = o[:, Hh:]
        else:
            outs[0][...] = o

    if mid:
        out_specs = [pl.BlockSpec((NB, Hh), lambda i: (i, 0)),
                     pl.BlockSpec((NB, Hh), lambda i: (i, 0))]
        out_shape = [jax.ShapeDtypeStruct((N, Hh), jnp.float32),
                     jax.ShapeDtypeStruct((N, Hh), jnp.float32)]
    else:
        out_specs = [pl.BlockSpec((NB, H), lambda i: (i, 0))]
        out_shape = [jax.ShapeDtypeStruct((N, H), jnp.float32)]

    res = pl.pallas_call(
        kfn,
        grid=(N // NB,),
        in_specs=[pl.BlockSpec((NB, Hh), lambda i: (i, 0)),
                  pl.BlockSpec((NB, Hh), lambda i: (i, 0)),
                  pl.BlockSpec((NB, Hh), lambda i: (i, 0)),
                  pl.BlockSpec((NB, Hh), lambda i: (i, 0)),
                  pl.BlockSpec((H, H), lambda i: (0, 0)),
                  pl.BlockSpec((1, H), lambda i: (0, 0)),
                  pl.BlockSpec((H, H), lambda i: (0, 0)),
                  pl.BlockSpec((1, H), lambda i: (0, 0))],
        out_specs=out_specs,
        out_shape=out_shape,
        interpret=_TC_INTERPRET,
    )(ha, hb, ga, gb, W1, b1, W2, b2)
    return res if mid else res[0]


def _sc_aggr(h_a, h_b, ea_a_pk, ea_b_pk, src2, dst2):
    N, Hh = h_a.shape
    nchunks, K = src2.shape
    T = 16
    base_chunks = nchunks // T
    extra = nchunks % T
    G = 32
    NGRP = -(-(base_chunks + 1) // G)
    RPT = 3200
    ZR = 80
    nz_last = (N - 15 * RPT) // ZR
    nz_main = RPT // ZR
    KP = K // 4

    mesh = plsc.VectorSubcoreMesh(core_axis_name="c", subcore_axis_name="s",
                                  num_cores=2, num_subcores=T)

    @functools.partial(
        pl.kernel,
        out_type=[jax.ShapeDtypeStruct((N, Hh), jnp.float32),
                  jax.ShapeDtypeStruct((N, Hh), jnp.float32)],
        mesh=mesh,
        scratch_types=[
            pltpu.VMEM((G, K), jnp.int32),
            pltpu.VMEM((G, K), jnp.int32),
            pltpu.VMEM((2, K, Hh), jnp.float32),
            pltpu.VMEM((2, KP, 128), jnp.float32),
            pltpu.VMEM((ZR, Hh), jnp.float32),
            pltpu.VMEM_SHARED((N, Hh), jnp.float32),
            pltpu.SemaphoreType.DMA,
            pltpu.SemaphoreType.DMA,
            pltpu.SemaphoreType.DMA,
            pltpu.SemaphoreType.DMA,
            pltpu.SemaphoreType.DMA,
            pltpu.SemaphoreType.DMA,
        ],
        compiler_params=pltpu.CompilerParams(use_tc_tiling_on_sc=False),
    )
    def k(ha_ref, hb_ref, eaa_ref, eab_ref, src_ref, dst_ref,
          outa_ref, outb_ref, sgrp, dgrp, gath, eabuf, stage, aggr,
          sg0, sg1, se0, se1, ss0, ss1):
        c = lax.axis_index("c")
        s = lax.axis_index("s")
        sem_g = (sg0, sg1)
        sem_e = (se0, se1)
        sem_s = (ss0, ss1)

        def body(h_ref, ea_ref, out_ref):
            def zrow(r, carry):
                z = jnp.zeros((16,), jnp.float32)
                for q in range(Hh // 16):
                    stage[r, pl.ds(q * 16, 16)] = z
                return carry
            lax.fori_loop(0, ZR, zrow, 0)
            row0 = s * RPT
            nzs = jnp.where(s < 15, nz_main, nz_last)

            def zchunk(t, carry):
                r0 = pl.multiple_of(row0 + t * ZR, 8)
                pltpu.sync_copy(stage, aggr.at[pl.ds(r0, ZR), :])
                return carry
            lax.fori_loop(0, nzs, zchunk, 0)
            plsc.subcore_barrier()

            start = s * base_chunks + jnp.minimum(s, extra)
            cnt = base_chunks + (s < extra).astype(jnp.int32)

            def gather_desc(slot, row):
                return pltpu.make_async_copy(
                    h_ref.at[sgrp.at[row]], gath.at[slot], sem_g[slot])

            def ea_desc(slot, j):
                p0 = pl.multiple_of(j * KP, KP)
                return pltpu.make_async_copy(
                    ea_ref.at[pl.ds(p0, KP), :], eabuf.at[slot],
                    sem_e[slot])

            def scat_desc(slot, row):
                return pltpu.make_async_copy(
                    gath.at[slot], aggr.at[dgrp.at[row]], sem_s[slot])

            def issue(slot, row, j, pend):
                @pl.when(pend == 1)
                def _():
                    scat_desc(slot, row).wait()
                gather_desc(slot, row).start()
                ea_desc(slot, j).start()

            def process(slot, row):
                gather_desc(slot, row).wait()
                ea_desc(slot, 0).wait()

                def prow(pr, carry):
                    for sub in range(8):
                        e = 4 * pr + sub // 2
                        f0 = (sub % 2) * 16
                        a = gath[slot, e, pl.ds(f0, 16)]
                        b = eabuf[slot, pr, pl.ds(sub * 16, 16)]
                        gath[slot, e, pl.ds(f0, 16)] = jnp.maximum(a + b, 0.0)
                    return carry
                lax.fori_loop(0, KP, prow, 0)
                scat_desc(slot, row).start(add=True)

            pend0 = jnp.int32(0)
            pend1 = jnp.int32(0)
            for g in range(NGRP):
                gbase = start + g * G
                gcnt = jnp.minimum(G, cnt - g * G)
                load_base = jnp.minimum(gbase, nchunks - G)
                roff = gbase - load_base
                pltpu.sync_copy(src_ref.at[pl.ds(load_base, G), :], sgrp)
                pltpu.sync_copy(dst_ref.at[pl.ds(load_base, G), :], dgrp)

                issue(0, roff, gbase, pend0)
                npairs = (gcnt + 1) >> 1

                def pair(p, carry):
                    p0, p1 = carry
                    jj0 = 2 * p
                    jj1 = jj0 + 1

                    @pl.when(jj1 < gcnt)
                    def _():
                        issue(1, roff + jj1, gbase + jj1, p1)
                    process(0, roff + jj0)
                    new_p0 = jnp.int32(1)

                    @pl.when(jj1 + 1 < gcnt)
                    def _():
                        issue(0, roff + jj1 + 1, gbase + jj1 + 1, new_p0)

                    @pl.when(jj1 < gcnt)
                    def _():
                        process(1, roff + jj1)
                    new_p1 = jnp.where(jj1 < gcnt, jnp.int32(1), p1)
                    return (new_p0, new_p1)

                pend0, pend1 = lax.fori_loop(0, npairs, pair, (pend0, pend1))

            @pl.when(pend0 == 1)
            def _():
                scat_desc(0, 0).wait()

            @pl.when(pend1 == 1)
            def _():
                scat_desc(1, 0).wait()

            plsc.subcore_barrier()

            def wchunk(t, carry):
                r0 = pl.multiple_of(row0 + t * ZR, 8)
                pltpu.sync_copy(aggr.at[pl.ds(r0, ZR), :], stage)
                pltpu.sync_copy(stage, out_ref.at[pl.ds(r0, ZR), :])
                return carry
            lax.fori_loop(0, nzs, wchunk, 0)

        @pl.when(c == 0)
        def _():
            body(ha_ref, eaa_ref, outa_ref)

        @pl.when(c == 1)
        def _():
            body(hb_ref, eab_ref, outb_ref)

    return k(h_a, h_b, ea_a_pk, ea_b_pk, src2, dst2)


def kernel(x, edge_index, edge_attr, W_in, b_in, W_e, b_e, W1, b1, W2, b2):
    E = edge_attr.shape[0]
    De = edge_attr.shape[1]
    H = W_e.shape[1]
    Hh = H // 2

    src2 = edge_index[0].reshape(E // 128, 128)
    dst2 = edge_index[1].reshape(E // 128, 128)
    b_in2 = b_in.reshape(1, -1)
    b12 = b1.reshape(1, -1)
    b22 = b2.reshape(1, -1)

    ea4 = edge_attr.reshape(E // 4, 4 * De)
    z = jnp.zeros((De, Hh), jnp.float32)
    Wea = W_e[:, :Hh]
    Web = W_e[:, Hh:]
    W4a = jnp.block([[Wea if i == j else z for j in range(4)]
                     for i in range(4)])
    W4b = jnp.block([[Web if i == j else z for j in range(4)]
                     for i in range(4)])
    b4a = jnp.tile(b_e[:Hh], 4).reshape(1, 4 * Hh)
    b4b = jnp.tile(b_e[Hh:], 4).reshape(1, 4 * Hh)

    h0a, h0b = _tc_inproj(x, W_in, b_in2)
    ea0_pk, ea1_pk = _tc_edgeproj(ea4, W4a, W4b, b4a, b4b)

    g0, g1 = _sc_aggr(h0a, h0b, ea0_pk, ea1_pk, src2, dst2)
    h1a, h1b = _tc_mlp(h0a, h0b, g0, g1, W1, b12, W2, b22, mid=True)

    g0, g1 = _sc_aggr(h1a, h1b, ea0_pk, ea1_pk, src2, dst2)
    out = _tc_mlp(h1a, h1b, g0, g1, W1, b12, W2, b22, mid=False)
    return out
